# Initial kernel scaffold; baseline (speedup 1.0000x reference)
import functools
import jax
import jax.numpy as jnp
from jax.experimental import pallas as pl
from jax.experimental.pallas import tpu as pltpu

K = 256
ROWS = 128
COLS = 32768
ROW_BLOCK = 32


def _topk_mask_kernel(x_ref, o_ref):
    x = x_ref[...]
    b = pltpu.bitcast(x, jnp.int32)
    s = jax.lax.shift_right_arithmetic(b, 31)
    u = b ^ (s & jnp.int32(0x7FFFFFFF))

    int_min = jnp.int32(-2147483648)

    def body(j, p):
        t = p + (jnp.int32(1) << (jnp.int32(30) - j))
        cnt = jnp.sum((u >= t).astype(jnp.float32), axis=1, keepdims=True)
        return jnp.where(cnt >= K, t, p)

    p0 = jnp.full((x.shape[0], 1), int_min, dtype=jnp.int32)
    p = jax.lax.fori_loop(0, 31, body, p0)

    o_ref[...] = jnp.where(u >= p, x, jnp.float32(-100.0))


def kernel(x):
    grid = ROWS // ROW_BLOCK
    return pl.pallas_call(
        _topk_mask_kernel,
        grid=(grid,),
        in_specs=[pl.BlockSpec((ROW_BLOCK, COLS), lambda i: (i, 0))],
        out_specs=pl.BlockSpec((ROW_BLOCK, COLS), lambda i: (i, 0)),
        out_shape=jax.ShapeDtypeStruct((ROWS, COLS), jnp.float32),
    )(x)


# speedup vs baseline: 19.2070x; 19.2070x over previous
import functools
import jax
import jax.numpy as jnp
from jax.experimental import pallas as pl
from jax.experimental.pallas import tpu as pltpu

K = 256
ROWS = 128
COLS = 32768
ROW_BLOCK = 32


def _topk_mask_kernel(x_ref, o_ref):
    x = x_ref[...]
    b = pltpu.bitcast(x, jnp.int32)
    s = jax.lax.shift_right_arithmetic(b, 31)
    u = b ^ (s & jnp.int32(0x7FFFFFFF))

    int_min = jnp.int32(-2147483648)

    def body(j, p):
        t = p + (jnp.int32(1) << (jnp.int32(30) - j))
        cnt = jnp.sum((u >= t).astype(jnp.float32), axis=1, keepdims=True)
        return jnp.where(cnt >= K, t, p)

    cnt0 = jnp.sum((u >= 0).astype(jnp.float32), axis=1, keepdims=True)
    p0 = jnp.where(cnt0 >= K, jnp.int32(0), int_min)
    p = jax.lax.fori_loop(0, 31, body, p0)

    o_ref[...] = jnp.where(u >= p, x, jnp.float32(-100.0))


def kernel(x):
    grid = ROWS // ROW_BLOCK
    return pl.pallas_call(
        _topk_mask_kernel,
        grid=(grid,),
        in_specs=[pl.BlockSpec((ROW_BLOCK, COLS), lambda i: (i, 0))],
        out_specs=pl.BlockSpec((ROW_BLOCK, COLS), lambda i: (i, 0)),
        out_shape=jax.ShapeDtypeStruct((ROWS, COLS), jnp.float32),
    )(x)
